# baseline (speedup 1.0000x reference)
import functools

import jax
import jax.numpy as jnp
from jax import lax
from jax.experimental import pallas as pl
from jax.experimental.pallas import tpu as pltpu
from jax.experimental.pallas import tpu_sc as plsc

NUM_WORKERS = 32
BBLK = 128
LANES = 16


def _make_kernel(b: int, l: int, embed: int):
    n_units = l * (b // BBLK)
    per_w = n_units // NUM_WORKERS
    n_groups = per_w // 2
    eblk = embed // 8
    bb_per_l = b // BBLK
    ubuf = embed * BBLK

    mesh = plsc.VectorSubcoreMesh(core_axis_name="c", subcore_axis_name="s")

    @functools.partial(
        pl.kernel,
        mesh=mesh,
        out_type=jax.ShapeDtypeStruct((n_units * ubuf,), jnp.float32),
        scratch_types=[
            pltpu.VMEM((per_w, BBLK), jnp.int32),
            pltpu.VMEM((2, BBLK, embed), jnp.float32),
            pltpu.VMEM((2, ubuf), jnp.float32),
            pltpu.SemaphoreType.DMA((2,)),
            pltpu.SemaphoreType.DMA((2,)),
        ],
        compiler_params=pltpu.CompilerParams(
            use_tc_tiling_on_sc=False, needs_layout_passes=False
        ),
    )
    def gather_kernel(qlin_hbm, table_hbm, out_hbm, idx_v, rows_v, buf_v,
                      gsem, wsem):
        wid = lax.axis_index("s") * 2 + lax.axis_index("c")
        u0 = wid * per_w

        pltpu.sync_copy(qlin_hbm.at[pl.ds(u0, per_w)], idx_v)

        def gather_start(slot, i):
            pltpu.async_copy(
                table_hbm.at[idx_v.at[i]],
                rows_v.at[slot],
                gsem.at[slot],
            )

        def gather_wait(slot):
            pltpu.make_async_copy(
                table_hbm.at[idx_v.at[0]],
                rows_v.at[slot],
                gsem.at[slot],
            ).wait()

        def write_wait(slot):
            for eb in range(eblk):
                pltpu.make_async_copy(
                    buf_v.at[slot, pl.ds(eb * 1024, 1024)],
                    out_hbm.at[pl.ds(0, 1024)],
                    wsem.at[slot],
                ).wait()

        bc_vecs = [
            lax.iota(jnp.int32, LANES) + seg * LANES
            for seg in range(BBLK // LANES)
        ]

        def transpose_unit(slot):
            for e in range(embed):
                e_vec = jnp.full((LANES,), e, jnp.int32)
                vals = [
                    plsc.load_gather(rows_v.at[slot], [bc_vecs[seg], e_vec])
                    for seg in range(BBLK // LANES)
                ]
                for seg in range(BBLK // LANES):
                    buf_v[slot, pl.ds(e * BBLK + seg * LANES, LANES)] = vals[seg]

        def write_start(slot, u):
            l_id = u // bb_per_l
            bb = u - l_id * bb_per_l
            for eb in range(eblk):
                base = ((l_id * eblk + eb) * bb_per_l + bb) * 1024
                pltpu.async_copy(
                    buf_v.at[slot, pl.ds(eb * 1024, 1024)],
                    out_hbm.at[pl.ds(base, 1024)],
                    wsem.at[slot],
                )

        gather_start(0, 0)
        gather_start(1, 1)

        def body(g, carry):
            i0 = g * 2
            for slot in range(2):
                i = i0 + slot
                gather_wait(slot)

                @pl.when(g > 0)
                def _():
                    write_wait(slot)

                transpose_unit(slot)
                write_start(slot, u0 + i)

                @pl.when(g + 1 < n_groups)
                def _():
                    gather_start(slot, i + 2)

            return carry

        lax.fori_loop(0, n_groups, body, 0)
        write_wait(0)
        write_wait(1)

    return gather_kernel


def kernel(q, W):
    b, l = q.shape
    _, embed = W.shape
    qlin = q.T.reshape(l * (b // BBLK), BBLK).astype(jnp.int32)
    out = _make_kernel(b, l, embed)(qlin, W)
    x5 = out.reshape(l, embed // 8, b // BBLK, 8, BBLK)
    return x5.transpose(2, 4, 0, 1, 3).reshape(b, l, embed)


# speedup vs baseline: 1.4716x; 1.4716x over previous
import functools

import jax
import jax.numpy as jnp
from jax import lax
from jax.experimental import pallas as pl
from jax.experimental.pallas import tpu as pltpu
from jax.experimental.pallas import tpu_sc as plsc

NUM_WORKERS = 32
BBLK = 128
BPAD = BBLK + 1
LANES = 16


def _make_kernel(b: int, l: int, embed: int):
    n_units = l * (b // BBLK)
    per_w = n_units // NUM_WORKERS
    n_groups = per_w // 2
    eblk = embed // 8
    bb_per_l = b // BBLK

    mesh = plsc.VectorSubcoreMesh(core_axis_name="c", subcore_axis_name="s")

    @functools.partial(
        pl.kernel,
        mesh=mesh,
        out_type=jax.ShapeDtypeStruct((n_units * embed, BBLK), jnp.float32),
        scratch_types=[
            pltpu.VMEM((per_w, BBLK), jnp.int32),
            pltpu.VMEM((2, BBLK, embed), jnp.float32),
            pltpu.VMEM((2, embed, BPAD), jnp.float32),
            pltpu.SemaphoreType.DMA((2,)),
            pltpu.SemaphoreType.DMA((2,)),
        ],
        compiler_params=pltpu.CompilerParams(
            use_tc_tiling_on_sc=False, needs_layout_passes=False
        ),
    )
    def gather_kernel(qlin_hbm, table_hbm, out_hbm, idx_v, rows_v, buf_v,
                      gsem, wsem):
        wid = lax.axis_index("s") * 2 + lax.axis_index("c")
        u0 = wid * per_w

        pltpu.sync_copy(qlin_hbm.at[pl.ds(u0, per_w)], idx_v)

        def gather_start(slot, i):
            pltpu.async_copy(
                table_hbm.at[idx_v.at[i]],
                rows_v.at[slot],
                gsem.at[slot],
            )

        def gather_wait(slot):
            pltpu.make_async_copy(
                table_hbm.at[idx_v.at[0]],
                rows_v.at[slot],
                gsem.at[slot],
            ).wait()

        def write_wait(slot):
            for eb in range(eblk):
                pltpu.make_async_copy(
                    buf_v.at[slot, pl.ds(eb * 8, 8), pl.ds(0, BBLK)],
                    out_hbm.at[pl.ds(0, 8)],
                    wsem.at[slot],
                ).wait()

        e_iotas = [
            lax.iota(jnp.int32, LANES) + k * LANES
            for k in range(embed // LANES)
        ]

        def transpose_unit(slot):
            for bc0 in range(0, BBLK, 8):
                for k in range(embed // LANES):
                    vals = [
                        rows_v[slot, bc0 + j, pl.ds(k * LANES, LANES)]
                        for j in range(8)
                    ]
                    for j in range(8):
                        plsc.store_scatter(
                            buf_v.at[slot],
                            [e_iotas[k], jnp.full((LANES,), bc0 + j, jnp.int32)],
                            vals[j],
                        )

        def write_start(slot, u):
            l_id = u // bb_per_l
            bb = u - l_id * bb_per_l
            for eb in range(eblk):
                base = ((l_id * eblk + eb) * bb_per_l + bb) * 8
                pltpu.async_copy(
                    buf_v.at[slot, pl.ds(eb * 8, 8), pl.ds(0, BBLK)],
                    out_hbm.at[pl.ds(base, 8)],
                    wsem.at[slot],
                )

        gather_start(0, 0)
        gather_start(1, 1)

        def body(g, carry):
            i0 = g * 2
            for slot in range(2):
                i = i0 + slot
                gather_wait(slot)

                @pl.when(g > 0)
                def _():
                    write_wait(slot)

                transpose_unit(slot)
                write_start(slot, u0 + i)

                @pl.when(g + 1 < n_groups)
                def _():
                    gather_start(slot, i + 2)

            return carry

        lax.fori_loop(0, n_groups, body, 0)
        write_wait(0)
        write_wait(1)

    return gather_kernel


def kernel(q, W):
    b, l = q.shape
    _, embed = W.shape
    qlin = q.T.reshape(l * (b // BBLK), BBLK).astype(jnp.int32)
    out = _make_kernel(b, l, embed)(qlin, W)
    x5 = out.reshape(l, embed // 8, b // BBLK, 8, BBLK)
    return x5.transpose(2, 4, 0, 1, 3).reshape(b, l, embed)
